# Initial kernel scaffold; baseline (speedup 1.0000x reference)
import jax
import jax.numpy as jnp
from jax.experimental import pallas as pl


def kernel(classifications, regressions, anchors, annotations):
    raise NotImplementedError("write your pallas kernel here")


# speedup vs baseline: 3.8982x; 3.8982x over previous
import functools

import jax
import jax.numpy as jnp
from jax import lax
from jax.experimental import pallas as pl
from jax.experimental.pallas import tpu as pltpu
from jax.experimental.pallas import tpu_sc as plsc

_ALPHA = 0.25
_LANES = 16


def _make_sc_match(B, M, A_pad, n_tiles):
    TPB = n_tiles // B
    CH = A_pad // TPB
    NV = CH // _LANES
    f32 = jnp.float32

    mesh = plsc.VectorSubcoreMesh(core_axis_name="c", subcore_axis_name="s")

    @functools.partial(
        pl.kernel,
        mesh=mesh,
        compiler_params=pltpu.CompilerParams(needs_layout_passes=False),
        out_type=[
            jax.ShapeDtypeStruct((B * A_pad,), f32),
            jax.ShapeDtypeStruct((B * A_pad,), f32),
            jax.ShapeDtypeStruct((B * 4 * A_pad,), f32),
        ],
        scratch_types=(
            [pltpu.VMEM((CH,), f32) for _ in range(4)]
            + [pltpu.VMEM((M * _LANES,), f32) for _ in range(6)]
            + [pltpu.VMEM((CH,), f32) for _ in range(2)]
        ),
    )
    def sc_match(ax1_h, ay1_h, ax2_h, ay2_h,
                 bx1_h, by1_h, bx2_h, by2_h, blab_h,
                 pos_o, lab_o, u_o,
                 ax1_v, ay1_v, ax2_v, ay2_v,
                 bx1_v, by1_v, bx2_v, by2_v, bar_v, blab_v,
                 pos_v, labo_v):
        wid = lax.axis_index("s") * 2 + lax.axis_index("c")
        b = wid // TPB
        part = wid % TPB
        abase = part * CH

        pltpu.sync_copy(ax1_h.at[pl.ds(abase, CH)], ax1_v)
        pltpu.sync_copy(ay1_h.at[pl.ds(abase, CH)], ay1_v)
        pltpu.sync_copy(ax2_h.at[pl.ds(abase, CH)], ax2_v)
        pltpu.sync_copy(ay2_h.at[pl.ds(abase, CH)], ay2_v)
        bbase = b * (M * _LANES)
        pltpu.sync_copy(bx1_h.at[pl.ds(bbase, M * _LANES)], bx1_v)
        pltpu.sync_copy(by1_h.at[pl.ds(bbase, M * _LANES)], by1_v)
        pltpu.sync_copy(bx2_h.at[pl.ds(bbase, M * _LANES)], bx2_v)
        pltpu.sync_copy(by2_h.at[pl.ds(bbase, M * _LANES)], by2_v)
        pltpu.sync_copy(blab_h.at[pl.ds(bbase, M * _LANES)], blab_v)

        for m in range(M):
            sl = pl.ds(m * _LANES, _LANES)
            bar_v[sl] = (bx2_v[sl] - bx1_v[sl]) * (by2_v[sl] - by1_v[sl])

        lanes = lax.iota(jnp.int32, _LANES)

        def step(v, carry):
            sl = pl.ds(v * _LANES, _LANES)
            ax1 = ax1_v[sl]
            ay1 = ay1_v[sl]
            ax2 = ax2_v[sl]
            ay2 = ay2_v[sl]
            aw = ax2 - ax1
            ah = ay2 - ay1
            aarea = aw * ah
            best = jnp.full((_LANES,), -1.0, f32)
            bidx = jnp.zeros((_LANES,), jnp.int32)
            for m in range(M):
                s2 = pl.ds(m * _LANES, _LANES)
                iw = jnp.minimum(ax2, bx2_v[s2]) - jnp.maximum(ax1, bx1_v[s2])
                ih = jnp.minimum(ay2, by2_v[s2]) - jnp.maximum(ay1, by1_v[s2])
                iw = jnp.maximum(iw, 0.0)
                ih = jnp.maximum(ih, 0.0)
                inter = iw * ih
                ua = jnp.maximum(aarea + bar_v[s2] - inter, 1e-8)
                iou = inter / ua
                gt = iou > best
                best = jnp.where(gt, iou, best)
                bidx = jnp.where(gt, m, bidx)
            pos = jnp.where(best >= 0.5, 1.0, 0.0)
            gidx = bidx * _LANES + lanes
            gx1 = plsc.load_gather(bx1_v, [gidx])
            gy1 = plsc.load_gather(by1_v, [gidx])
            gx2 = plsc.load_gather(bx2_v, [gidx])
            gy2 = plsc.load_gather(by2_v, [gidx])
            glab = plsc.load_gather(blab_v, [gidx])
            gw = gx2 - gx1
            gh = gy2 - gy1
            gcx = gx1 + 0.5 * gw
            gcy = gy1 + 0.5 * gh
            acx = ax1 + 0.5 * aw
            acy = ay1 + 0.5 * ah
            u0 = (gcx - acx) / aw * 10.0
            u1 = (gcy - acy) / ah * 10.0
            u2 = jnp.maximum(gw, 1.0) / aw
            u3 = jnp.maximum(gh, 1.0) / ah
            pos_v[sl] = pos
            labo_v[sl] = glab
            ax1_v[sl] = u0
            ay1_v[sl] = u1
            ax2_v[sl] = u2
            ay2_v[sl] = u3
            return carry

        lax.fori_loop(0, NV, step, 0)

        obase = b * A_pad + abase
        pltpu.sync_copy(pos_v, pos_o.at[pl.ds(obase, CH)])
        pltpu.sync_copy(labo_v, lab_o.at[pl.ds(obase, CH)])
        for c, src in enumerate((ax1_v, ay1_v, ax2_v, ay2_v)):
            pltpu.sync_copy(src, u_o.at[pl.ds((b * 4 + c) * A_pad + abase, CH)])

    return sc_match


def _reg_body(u_ref, r_ref, p_ref, out_ref, *, B):
    b = pl.program_id(0)
    u = u_ref[0]
    r = r_ref[0]
    p = p_ref[0]
    is_lin = lax.broadcasted_iota(jnp.int32, (4, 1), 0) < 2
    t = jnp.where(is_lin, u, 5.0 * jnp.log(u))
    d = jnp.abs(t - r)
    sl1 = jnp.where(d <= 1.0 / 9.0, 4.5 * d * d, d - 0.5 / 9.0)
    reg_sum = jnp.sum(sl1 * p)
    npos = jnp.sum(p)

    @pl.when(b == 0)
    def _():
        out_ref[...] = jnp.zeros_like(out_ref)

    row = lax.broadcasted_iota(jnp.int32, (2, B), 1) == b
    vals = jnp.concatenate(
        [jnp.full((1, B), reg_sum), jnp.full((1, B), npos)], axis=0)
    out_ref[...] += jnp.where(row, vals, 0.0)


def _focal_body(cls_ref, pos_ref, lab_ref, rn_ref, out_ref, acc_ref, *, B, K, C):
    b = pl.program_id(0)
    k = pl.program_id(1)

    @pl.when(k == 0)
    def _():
        acc_ref[0] = 0.0

    @pl.when((k == 0) & (b == 0))
    def _():
        acc_ref[1] = 0.0
        acc_ref[2] = 0.0

    X = cls_ref[0]
    P = pos_ref[0]
    L = lab_ref[0]
    cls_iota = lax.broadcasted_iota(jnp.int32, X.shape, 1).astype(jnp.float32)
    mask = (P < 0.5) | (cls_iota == L)
    x = jnp.clip(X, 1e-4, 1.0 - 1e-4)
    s = jnp.where(mask, -x, x)
    e = jnp.exp(-s)
    softplus = s + jnp.log(1.0 + e)
    inv = 1.0 / (1.0 + e)
    w = jnp.where(mask, _ALPHA, 1.0 - _ALPHA)
    loss = w * (inv * inv) * softplus
    acc_ref[0] += jnp.sum(loss)

    @pl.when(k == K - 1)
    def _():
        reg_sum = rn_ref[0, b]
        npos = rn_ref[1, b]
        cls_img = acc_ref[0] / jnp.maximum(npos, 0.01)
        reg_img = jnp.where(npos > 0.0,
                            reg_sum / jnp.maximum(4.0 * npos, 1.0), 0.0)
        acc_ref[1] += cls_img
        acc_ref[2] += reg_img

        @pl.when(b == B - 1)
        def _():
            out_ref[0] = acc_ref[1] / B
            out_ref[1] = acc_ref[2] / B


def kernel(classifications, regressions, anchors, annotations):
    B, A, C = classifications.shape
    M = annotations.shape[1]
    f32 = jnp.float32

    n_tiles = 32
    TPB = n_tiles // B
    CH = -(-A // (TPB * _LANES)) * _LANES
    A_pad = CH * TPB

    a = anchors[0].astype(f32)
    pad_n = A_pad - A
    ax1 = jnp.pad(a[:, 0], (0, pad_n))
    ay1 = jnp.pad(a[:, 1], (0, pad_n))
    ax2 = jnp.pad(a[:, 2], (0, pad_n), constant_values=1.0)
    ay2 = jnp.pad(a[:, 3], (0, pad_n), constant_values=1.0)

    ann = annotations.astype(f32)
    def bcast(col):
        return jnp.broadcast_to(ann[:, :, col:col + 1], (B, M, _LANES)).reshape(-1)
    bx1, by1, bx2, by2, blab = (bcast(c) for c in range(5))

    sc_match = _make_sc_match(B, M, A_pad, n_tiles)
    pos_f, lab_f, u_f = sc_match(ax1, ay1, ax2, ay2, bx1, by1, bx2, by2, blab)

    pos2 = pos_f.reshape(B, A_pad)
    u3 = u_f.reshape(B, 4, A_pad)
    pos_col = pos2[:, :A].reshape(B, A, 1)
    lab_col = lab_f.reshape(B, A_pad)[:, :A].reshape(B, A, 1)

    regs_p = jnp.pad(regressions.astype(f32).transpose(0, 2, 1),
                     ((0, 0), (0, 0), (0, pad_n)))

    regnp = pl.pallas_call(
        functools.partial(_reg_body, B=B),
        grid=(B,),
        in_specs=[
            pl.BlockSpec((1, 4, A_pad), lambda b: (b, 0, 0)),
            pl.BlockSpec((1, 4, A_pad), lambda b: (b, 0, 0)),
            pl.BlockSpec((1, 1, A_pad), lambda b: (b, 0, 0)),
        ],
        out_specs=pl.BlockSpec((2, B), lambda b: (0, 0)),
        out_shape=jax.ShapeDtypeStruct((2, B), f32),
    )(u3, regs_p, pos2.reshape(B, 1, A_pad))

    A_blk = 1584
    assert A % A_blk == 0
    K = A // A_blk

    out2 = pl.pallas_call(
        functools.partial(_focal_body, B=B, K=K, C=C),
        grid=(B, K),
        in_specs=[
            pl.BlockSpec((1, A_blk, C), lambda b, k: (b, k, 0)),
            pl.BlockSpec((1, A_blk, 1), lambda b, k: (b, k, 0)),
            pl.BlockSpec((1, A_blk, 1), lambda b, k: (b, k, 0)),
            pl.BlockSpec(memory_space=pltpu.SMEM),
        ],
        out_specs=pl.BlockSpec(memory_space=pltpu.SMEM),
        out_shape=jax.ShapeDtypeStruct((2,), f32),
        scratch_shapes=[pltpu.SMEM((4,), f32)],
    )(classifications.astype(f32), pos_col, lab_col, regnp)

    return (out2[0:1], out2[1:2])
